# Initial kernel scaffold; baseline (speedup 1.0000x reference)
import functools

import jax
import jax.numpy as jnp
from jax import lax
from jax.experimental import pallas as pl
from jax.experimental.pallas import tpu as pltpu
from jax.experimental.pallas import tpu_sc as plsc

L = 16
NC = 2
NS = 16

KW_V = 50000
AV_V = 100000
KW_TAB = 50176
AV_TAB = 100352
KW_CH = 25
AV_CH = 49
TOK_ROWS = 50


def _build_sc_kernel():
    mesh = plsc.VectorSubcoreMesh(core_axis_name="c", subcore_axis_name="s")

    @functools.partial(
        pl.kernel,
        out_type=[
            jax.ShapeDtypeStruct((1600, 128), jnp.int32),
            jax.ShapeDtypeStruct((32, 128), jnp.int32),
        ],
        mesh=mesh,
        scratch_types=[
            pltpu.VMEM_SHARED((KW_TAB,), jnp.int32),
            pltpu.VMEM_SHARED((AV_TAB,), jnp.int32),
            pltpu.VMEM((AV_CH * 128,), jnp.int32),
            pltpu.VMEM((AV_CH, 128), jnp.int32),
            pltpu.VMEM((AV_CH, 128), jnp.int32),
            pltpu.VMEM((TOK_ROWS, 128), jnp.int32),
            pltpu.VMEM((TOK_ROWS, 128), jnp.int32),
            pltpu.VMEM((1, 128), jnp.int32),
            pltpu.VMEM((1, 128), jnp.int32),
        ],
    )
    def k(tok_hbm, art_hbm, kw_hbm, av_hbm, otok_hbm, oart_hbm,
          kwt_sh, avt_sh, raw_v, idx_v, val_v, tok_v, toko_v, art_v, arto_v):
        s = lax.axis_index("s")
        wid = lax.axis_index("c") * NS + s
        iota = lax.broadcasted_iota(jnp.int32, (L,), 0)

        def fill_zero(i, carry):
            raw_v[pl.ds(i * L, L)] = jnp.zeros((L,), jnp.int32)
            return carry
        lax.fori_loop(0, (AV_TAB // NS) // L, fill_zero, 0)
        pltpu.sync_copy(raw_v.at[pl.ds(0, AV_TAB // NS)],
                        avt_sh.at[pl.ds(s * (AV_TAB // NS), AV_TAB // NS)])

        def fill_one(i, carry):
            raw_v[pl.ds(i * L, L)] = jnp.ones((L,), jnp.int32)
            return carry
        lax.fori_loop(0, (KW_TAB // NS) // L, fill_one, 0)
        pltpu.sync_copy(raw_v.at[pl.ds(0, KW_TAB // NS)],
                        kwt_sh.at[pl.ds(s * (KW_TAB // NS), KW_TAB // NS)])

        plsc.subcore_barrier()

        def scatter_vocab(vocab_hbm, table_sh, n_chunks, vocab_n, query_max, off):
            n = n_chunks * 128
            base = jnp.minimum(s * n, vocab_n - n)
            pltpu.sync_copy(vocab_hbm.at[pl.ds(base, n)], raw_v.at[pl.ds(0, n)])

            def prep(t, carry):
                v = raw_v[pl.ds(t * L, L)]
                j = t // 8
                o = (t % 8) * L
                idx_v[j, pl.ds(o, L)] = jnp.where(v < query_max, v, query_max)
                val_v[j, pl.ds(o, L)] = base + t * L + iota + off
                return carry
            lax.fori_loop(0, n // L, prep, 0)

            def scat(j, carry):
                pltpu.sync_copy(val_v.at[j], table_sh.at[idx_v.at[j]])
                return carry
            lax.fori_loop(0, n_chunks, scat, 0)

        scatter_vocab(kw_hbm, kwt_sh, KW_CH, KW_V, KW_V, 2)
        scatter_vocab(av_hbm, avt_sh, AV_CH, AV_V, AV_V, 1)

        plsc.subcore_barrier()

        pltpu.sync_copy(tok_hbm.at[pl.ds(wid * TOK_ROWS, TOK_ROWS)], tok_v)

        def tok_gather(j, carry):
            pltpu.sync_copy(kwt_sh.at[tok_v.at[j]], toko_v.at[j])
            return carry
        lax.fori_loop(0, TOK_ROWS, tok_gather, 0)
        pltpu.sync_copy(toko_v, otok_hbm.at[pl.ds(wid * TOK_ROWS, TOK_ROWS)])

        pltpu.sync_copy(art_hbm.at[pl.ds(wid, 1)], art_v)
        pltpu.sync_copy(avt_sh.at[art_v.at[0]], arto_v.at[0])
        pltpu.sync_copy(arto_v, oart_hbm.at[pl.ds(wid, 1)])

    return k


_sc_lookup = _build_sc_kernel()


@jax.jit
def kernel(token_ids, article_ids, keyword_vocab, article_vocab):
    tok2d = token_ids.reshape(1600, 128)
    art2d = article_ids.reshape(32, 128)
    otok, oart = _sc_lookup(tok2d, art2d, keyword_vocab, article_vocab)
    return otok.reshape(4096, 50), oart.reshape(4096)


# speedup vs baseline: 312.7025x; 312.7025x over previous
import functools

import jax
import jax.numpy as jnp
from jax import lax
from jax.experimental import pallas as pl
from jax.experimental.pallas import tpu as pltpu
from jax.experimental.pallas import tpu_sc as plsc

L = 16
NC = 2
NS = 16

KW_V = 50000
AV_V = 100000
KW_TAB = 50176
AV_TAB = 100352
KW_CH = 25
AV_CH = 49
TOK_ROWS = 50


def _build_sc_kernel():
    mesh = plsc.VectorSubcoreMesh(core_axis_name="c", subcore_axis_name="s")

    @functools.partial(
        pl.kernel,
        out_type=[
            jax.ShapeDtypeStruct((32, TOK_ROWS, 128), jnp.int32),
            jax.ShapeDtypeStruct((32, 1, 128), jnp.int32),
        ],
        mesh=mesh,
        scratch_types=[
            pltpu.VMEM_SHARED((KW_TAB,), jnp.int32),
            pltpu.VMEM_SHARED((AV_TAB,), jnp.int32),
            pltpu.VMEM((AV_CH * 128,), jnp.int32),
            pltpu.VMEM((AV_CH, 128), jnp.int32),
            pltpu.VMEM((AV_CH, 128), jnp.int32),
            pltpu.VMEM((TOK_ROWS, 128), jnp.int32),
            pltpu.VMEM((TOK_ROWS, 128), jnp.int32),
            pltpu.VMEM((1, 128), jnp.int32),
            pltpu.VMEM((1, 128), jnp.int32),
        ],
    )
    def k(tok_hbm, art_hbm, kw_hbm, av_hbm, otok_hbm, oart_hbm,
          kwt_sh, avt_sh, raw_v, idx_v, val_v, tok_v, toko_v, art_v, arto_v):
        s = lax.axis_index("s")
        wid = lax.axis_index("c") * NS + s
        iota = lax.broadcasted_iota(jnp.int32, (L,), 0)

        def fill_zero(i, carry):
            raw_v[pl.ds(i * L, L)] = jnp.zeros((L,), jnp.int32)
            return carry
        lax.fori_loop(0, (AV_TAB // NS) // L, fill_zero, 0)
        pltpu.sync_copy(raw_v.at[pl.ds(0, AV_TAB // NS)],
                        avt_sh.at[pl.ds(s * (AV_TAB // NS), AV_TAB // NS)])

        def fill_one(i, carry):
            raw_v[pl.ds(i * L, L)] = jnp.ones((L,), jnp.int32)
            return carry
        lax.fori_loop(0, (KW_TAB // NS) // L, fill_one, 0)
        pltpu.sync_copy(raw_v.at[pl.ds(0, KW_TAB // NS)],
                        kwt_sh.at[pl.ds(s * (KW_TAB // NS), KW_TAB // NS)])

        plsc.subcore_barrier()

        def scatter_vocab(vocab_hbm, table_sh, n_chunks, vocab_n, query_max, off):
            n = n_chunks * 128
            base = jnp.minimum(s * n, vocab_n - n)
            pltpu.sync_copy(vocab_hbm.at[pl.ds(base, n)], raw_v.at[pl.ds(0, n)])

            def prep(t, carry):
                v = raw_v[pl.ds(t * L, L)]
                j = t // 8
                o = (t % 8) * L
                idx_v[j, pl.ds(o, L)] = jnp.where(v < query_max, v, query_max)
                val_v[j, pl.ds(o, L)] = base + t * L + iota + off
                return carry
            lax.fori_loop(0, n // L, prep, 0)

            def scat(j, carry):
                pltpu.sync_copy(val_v.at[j], table_sh.at[idx_v.at[j]])
                return carry
            lax.fori_loop(0, n_chunks, scat, 0)

        scatter_vocab(kw_hbm, kwt_sh, KW_CH, KW_V, KW_V, 2)
        scatter_vocab(av_hbm, avt_sh, AV_CH, AV_V, AV_V, 1)

        plsc.subcore_barrier()

        pltpu.sync_copy(tok_hbm.at[wid], tok_v)

        def tok_gather(j, carry):
            pltpu.sync_copy(kwt_sh.at[tok_v.at[j]], toko_v.at[j])
            return carry
        lax.fori_loop(0, TOK_ROWS, tok_gather, 0)
        pltpu.sync_copy(toko_v, otok_hbm.at[wid])

        pltpu.sync_copy(art_hbm.at[wid], art_v)
        pltpu.sync_copy(avt_sh.at[art_v.at[0]], arto_v.at[0])
        pltpu.sync_copy(arto_v, oart_hbm.at[wid])

    return k


_sc_lookup = _build_sc_kernel()


@jax.jit
def kernel(token_ids, article_ids, keyword_vocab, article_vocab):
    tok3d = token_ids.reshape(32, TOK_ROWS, 128)
    art3d = article_ids.reshape(32, 1, 128)
    otok, oart = _sc_lookup(tok3d, art3d, keyword_vocab, article_vocab)
    return otok.reshape(4096, 50), oart.reshape(4096)


# speedup vs baseline: 585.5528x; 1.8726x over previous
import functools

import jax
import jax.numpy as jnp
from jax import lax
from jax.experimental import pallas as pl
from jax.experimental.pallas import tpu as pltpu
from jax.experimental.pallas import tpu_sc as plsc

L = 16
NC = 2
NS = 16

KW_V = 50000
AV_V = 100000
KW_PAD = 51200
AV_PAD = 100352
KW_ROWS = 25
AV_ROWS = 49
KW_TAB = 100352
AV_TAB = 200704
KW_INIT = 50176
AV_INIT = 100352
TOK_ROWS = 50

KW_SENT = KW_TAB - 1
AV_SENT = AV_TAB - 1


def _build_sc_kernel():
    mesh = plsc.VectorSubcoreMesh(core_axis_name="c", subcore_axis_name="s")

    @functools.partial(
        pl.kernel,
        out_type=[
            jax.ShapeDtypeStruct((32, TOK_ROWS, 128), jnp.int32),
            jax.ShapeDtypeStruct((32, 1, 128), jnp.int32),
        ],
        mesh=mesh,
        scratch_types=[
            pltpu.VMEM_SHARED((KW_TAB,), jnp.int32),
            pltpu.VMEM_SHARED((AV_TAB,), jnp.int32),
            pltpu.VMEM((KW_ROWS, 128), jnp.int32),
            pltpu.VMEM((KW_ROWS, 128), jnp.int32),
            pltpu.VMEM((AV_ROWS, 128), jnp.int32),
            pltpu.VMEM((AV_ROWS, 128), jnp.int32),
            pltpu.VMEM((TOK_ROWS, 128), jnp.int32),
            pltpu.VMEM((TOK_ROWS, 128), jnp.int32),
            pltpu.VMEM((1, 128), jnp.int32),
            pltpu.VMEM((1, 128), jnp.int32),
            pltpu.VMEM((KW_INIT // NS,), jnp.int32),
            pltpu.VMEM((AV_INIT // NS,), jnp.int32),
            pltpu.SemaphoreType.DMA,
            pltpu.SemaphoreType.DMA,
            pltpu.SemaphoreType.DMA,
        ],
    )
    def k(tok_hbm, art_hbm, kw_hbm, av_hbm, kwv_hbm, avv_hbm, ones_hbm,
          zeros_hbm, otok_hbm, oart_hbm,
          kwt_sh, avt_sh, kidx_v, kval_v, aidx_v, aval_v,
          tok_v, toko_v, art_v, arto_v, ones_v, zeros_v,
          ld_sem, init_sem, sc_sem):
        s = lax.axis_index("s")
        wid = lax.axis_index("c") * NS + s

        cp = pltpu.async_copy
        loads = [
            cp(kw_hbm.at[s], kidx_v, ld_sem),
            cp(kwv_hbm.at[s], kval_v, ld_sem),
            cp(av_hbm.at[s], aidx_v, ld_sem),
            cp(avv_hbm.at[s], aval_v, ld_sem),
            cp(tok_hbm.at[wid], tok_v, ld_sem),
            cp(art_hbm.at[wid], art_v, ld_sem),
        ]
        st1 = cp(ones_hbm, ones_v, init_sem)
        st2 = cp(zeros_hbm, zeros_v, init_sem)
        st1.wait()
        st2.wait()
        inits = [
            cp(ones_v,
               kwt_sh.at[pl.ds(s * (KW_INIT // NS), KW_INIT // NS)], init_sem),
            cp(zeros_v,
               avt_sh.at[pl.ds(s * (AV_INIT // NS), AV_INIT // NS)], init_sem),
        ]
        for c_ in loads:
            c_.wait()
        for c_ in inits:
            c_.wait()
        plsc.subcore_barrier()

        scats = [cp(kval_v.at[j], kwt_sh.at[kidx_v.at[j]], sc_sem)
                 for j in range(KW_ROWS)]
        scats += [cp(aval_v.at[j], avt_sh.at[aidx_v.at[j]], sc_sem)
                  for j in range(AV_ROWS)]
        for c_ in scats:
            c_.wait()
        plsc.subcore_barrier()

        gats = [cp(kwt_sh.at[tok_v.at[j]], toko_v.at[j], ld_sem)
                for j in range(TOK_ROWS)]
        gats.append(cp(avt_sh.at[art_v.at[0]], arto_v.at[0], ld_sem))
        for c_ in gats:
            c_.wait()
        pltpu.sync_copy(toko_v, otok_hbm.at[wid])
        pltpu.sync_copy(arto_v, oart_hbm.at[wid])

    return k


_sc_lookup = _build_sc_kernel()


@jax.jit
def kernel(token_ids, article_ids, keyword_vocab, article_vocab):
    tok3d = token_ids.reshape(32, TOK_ROWS, 128)
    art3d = article_ids.reshape(32, 1, 128)
    kwv = jnp.pad(keyword_vocab, (0, KW_PAD - KW_V),
                  constant_values=KW_SENT).reshape(NS, KW_ROWS, 128)
    avv = jnp.pad(article_vocab, (0, AV_PAD - AV_V),
                  constant_values=AV_SENT).reshape(NS, AV_ROWS, 128)
    kw_ramp = (jnp.arange(KW_PAD, dtype=jnp.int32) + 2).reshape(NS, KW_ROWS, 128)
    av_ramp = (jnp.arange(AV_PAD, dtype=jnp.int32) + 1).reshape(NS, AV_ROWS, 128)
    ones = jnp.ones((KW_INIT // NS,), jnp.int32)
    zeros = jnp.zeros((AV_INIT // NS,), jnp.int32)
    otok, oart = _sc_lookup(tok3d, art3d, kwv, avv, kw_ramp, av_ramp,
                            ones, zeros)
    return otok.reshape(4096, 50), oart.reshape(4096)


# speedup vs baseline: 588.2659x; 1.0046x over previous
import functools

import jax
import jax.numpy as jnp
from jax import lax
from jax.experimental import pallas as pl
from jax.experimental.pallas import tpu as pltpu
from jax.experimental.pallas import tpu_sc as plsc

L = 16
NC = 2
NS = 16

KW_V = 50000
AV_V = 100000
KW_PAD = 51200
AV_PAD = 100352
KW_ROWS = 25
AV_ROWS = 49
KW_TAB = 100352
AV_TAB = 200704
KW_INIT = 50176
AV_INIT = 100352
TOK_ROWS = 50

KW_SENT = KW_TAB - 1
AV_SENT = AV_TAB - 1


def _build_sc_kernel():
    mesh = plsc.VectorSubcoreMesh(core_axis_name="c", subcore_axis_name="s")

    @functools.partial(
        pl.kernel,
        out_type=[
            jax.ShapeDtypeStruct((32, TOK_ROWS, 128), jnp.int32),
            jax.ShapeDtypeStruct((32, 1, 128), jnp.int32),
        ],
        mesh=mesh,
        scratch_types=[
            pltpu.VMEM_SHARED((KW_TAB,), jnp.int32),
            pltpu.VMEM_SHARED((AV_TAB,), jnp.int32),
            pltpu.VMEM((KW_ROWS, 128), jnp.int32),
            pltpu.VMEM((KW_ROWS, 128), jnp.int32),
            pltpu.VMEM((AV_ROWS, 128), jnp.int32),
            pltpu.VMEM((AV_ROWS, 128), jnp.int32),
            pltpu.VMEM((TOK_ROWS, 128), jnp.int32),
            pltpu.VMEM((TOK_ROWS, 128), jnp.int32),
            pltpu.VMEM((1, 128), jnp.int32),
            pltpu.VMEM((1, 128), jnp.int32),
            pltpu.VMEM((KW_INIT // NS,), jnp.int32),
            pltpu.VMEM((AV_INIT // NS,), jnp.int32),
            pltpu.SemaphoreType.DMA,
            pltpu.SemaphoreType.DMA,
            pltpu.SemaphoreType.DMA,
            pltpu.SemaphoreType.DMA,
            pltpu.SemaphoreType.DMA,
            pltpu.SemaphoreType.DMA,
            pltpu.SemaphoreType.DMA,
            pltpu.SemaphoreType.DMA,
        ],
    )
    def k(tok_hbm, art_hbm, kw_hbm, av_hbm, kwv_hbm, avv_hbm, ones_hbm,
          zeros_hbm, otok_hbm, oart_hbm,
          kwt_sh, avt_sh, kidx_v, kval_v, aidx_v, aval_v,
          tok_v, toko_v, art_v, arto_v, ones_v, zeros_v,
          s_kw, s_av, s_q, s_stage, s_init, s_sck, s_sca, s_g):
        s = lax.axis_index("s")
        wid = lax.axis_index("c") * NS + s

        cp = pltpu.async_copy
        ld_kw = [cp(kw_hbm.at[s], kidx_v, s_kw),
                 cp(kwv_hbm.at[s], kval_v, s_kw)]
        ld_av = [cp(av_hbm.at[s], aidx_v, s_av),
                 cp(avv_hbm.at[s], aval_v, s_av)]
        ld_q = [cp(tok_hbm.at[wid], tok_v, s_q),
                cp(art_hbm.at[wid], art_v, s_q)]
        st1 = cp(ones_hbm, ones_v, s_stage)
        st2 = cp(zeros_hbm, zeros_v, s_stage)
        st1.wait()
        st2.wait()
        inits = [
            cp(ones_v,
               kwt_sh.at[pl.ds(s * (KW_INIT // NS), KW_INIT // NS)], s_init),
            cp(zeros_v,
               avt_sh.at[pl.ds(s * (AV_INIT // NS), AV_INIT // NS)], s_init),
        ]
        for c_ in inits:
            c_.wait()
        plsc.subcore_barrier()

        for c_ in ld_kw:
            c_.wait()
        kw_scats = [cp(kval_v.at[j], kwt_sh.at[kidx_v.at[j]], s_sck)
                    for j in range(KW_ROWS)]
        for c_ in ld_av:
            c_.wait()
        av_scats = [cp(aval_v.at[j], avt_sh.at[aidx_v.at[j]], s_sca)
                    for j in range(AV_ROWS)]
        for c_ in kw_scats:
            c_.wait()
        plsc.subcore_barrier()

        for c_ in ld_q:
            c_.wait()
        gats = [cp(kwt_sh.at[tok_v.at[j]], toko_v.at[j], s_g)
                for j in range(TOK_ROWS)]
        for c_ in av_scats:
            c_.wait()
        plsc.subcore_barrier()
        gats.append(cp(avt_sh.at[art_v.at[0]], arto_v.at[0], s_g))
        for c_ in gats:
            c_.wait()
        pltpu.sync_copy(toko_v, otok_hbm.at[wid])
        pltpu.sync_copy(arto_v, oart_hbm.at[wid])

    return k


_sc_lookup = _build_sc_kernel()


@jax.jit
def kernel(token_ids, article_ids, keyword_vocab, article_vocab):
    tok3d = token_ids.reshape(32, TOK_ROWS, 128)
    art3d = article_ids.reshape(32, 1, 128)
    kwv = jnp.pad(keyword_vocab, (0, KW_PAD - KW_V),
                  constant_values=KW_SENT).reshape(NS, KW_ROWS, 128)
    avv = jnp.pad(article_vocab, (0, AV_PAD - AV_V),
                  constant_values=AV_SENT).reshape(NS, AV_ROWS, 128)
    kw_ramp = (jnp.arange(KW_PAD, dtype=jnp.int32) + 2).reshape(NS, KW_ROWS, 128)
    av_ramp = (jnp.arange(AV_PAD, dtype=jnp.int32) + 1).reshape(NS, AV_ROWS, 128)
    ones = jnp.ones((KW_INIT // NS,), jnp.int32)
    zeros = jnp.zeros((AV_INIT // NS,), jnp.int32)
    otok, oart = _sc_lookup(tok3d, art3d, kwv, avv, kw_ramp, av_ramp,
                            ones, zeros)
    return otok.reshape(4096, 50), oart.reshape(4096)


# speedup vs baseline: 616.6597x; 1.0483x over previous
import functools

import jax
import jax.numpy as jnp
from jax import lax
from jax.experimental import pallas as pl
from jax.experimental.pallas import tpu as pltpu
from jax.experimental.pallas import tpu_sc as plsc

L = 16
NC = 2
NS = 16

KW_V = 50000
AV_V = 100000
KW_PAD = 51200
AV_PAD = 100352
KW_ROWS = 25
AV_ROWS = 49
KW_TAB = 100352
AV_TAB = 200704
KW_INIT = 50176
AV_INIT = 100352
TOK_ROWS = 50

KW_SENT = KW_TAB - 1
AV_SENT = AV_TAB - 1


def _build_sc_kernel():
    mesh = plsc.VectorSubcoreMesh(core_axis_name="c", subcore_axis_name="s")

    @functools.partial(
        pl.kernel,
        out_type=[
            jax.ShapeDtypeStruct((32, TOK_ROWS, 128), jnp.int32),
            jax.ShapeDtypeStruct((32, 1, 128), jnp.int32),
        ],
        mesh=mesh,
        scratch_types=[
            pltpu.VMEM_SHARED((KW_TAB,), jnp.int32),
            pltpu.VMEM_SHARED((AV_TAB,), jnp.int32),
            pltpu.VMEM((KW_ROWS, 128), jnp.int32),
            pltpu.VMEM((KW_ROWS, 128), jnp.int32),
            pltpu.VMEM((AV_ROWS, 128), jnp.int32),
            pltpu.VMEM((AV_ROWS, 128), jnp.int32),
            pltpu.VMEM((TOK_ROWS, 128), jnp.int32),
            pltpu.VMEM((TOK_ROWS, 128), jnp.int32),
            pltpu.VMEM((1, 128), jnp.int32),
            pltpu.VMEM((1, 128), jnp.int32),
            pltpu.VMEM((KW_INIT // NS,), jnp.int32),
            pltpu.VMEM((AV_INIT // NS,), jnp.int32),
            pltpu.SemaphoreType.DMA,
            pltpu.SemaphoreType.DMA,
            pltpu.SemaphoreType.DMA,
            pltpu.SemaphoreType.DMA,
            pltpu.SemaphoreType.DMA,
            pltpu.SemaphoreType.DMA,
            pltpu.SemaphoreType.DMA,
            pltpu.SemaphoreType.DMA,
        ],
    )
    def k(tok_hbm, art_hbm, kw_hbm, av_hbm, kwv_hbm, avv_hbm, ones_hbm,
          zeros_hbm, otok_hbm, oart_hbm,
          kwt_sh, avt_sh, kidx_v, kval_v, aidx_v, aval_v,
          tok_v, toko_v, art_v, arto_v, ones_v, zeros_v,
          s_kw, s_av, s_q, s_stage, s_init, s_sck, s_sca, s_g):
        s = lax.axis_index("s")
        wid = lax.axis_index("c") * NS + s

        cp = pltpu.async_copy
        ld_kw = [cp(kw_hbm.at[s], kidx_v, s_kw),
                 cp(kwv_hbm.at[s], kval_v, s_kw)]
        ld_av = [cp(av_hbm.at[s], aidx_v, s_av),
                 cp(avv_hbm.at[s], aval_v, s_av)]
        ld_q = [cp(tok_hbm.at[wid], tok_v, s_q),
                cp(art_hbm.at[wid], art_v, s_q)]
        st1 = cp(ones_hbm, ones_v, s_stage)
        st2 = cp(zeros_hbm, zeros_v, s_stage)
        st1.wait()
        st2.wait()
        inits = [
            cp(ones_v,
               kwt_sh.at[pl.ds(s * (KW_INIT // NS), KW_INIT // NS)], s_init),
            cp(zeros_v,
               avt_sh.at[pl.ds(s * (AV_INIT // NS), AV_INIT // NS)], s_init),
        ]
        for c_ in inits:
            c_.wait()
        plsc.subcore_barrier()

        def kept_rows(idx_ref, n_rows, bound):
            def cnt(j, acc):
                head = idx_ref[j, pl.ds(0, L)][0]
                return acc + (head < bound).astype(jnp.int32)
            return lax.fori_loop(0, n_rows, cnt, jnp.int32(0))

        for c_ in ld_kw:
            c_.wait()
        k_kw = kept_rows(kidx_v, KW_ROWS, KW_V)

        def kw_fire(j, carry):
            cp(kval_v.at[j], kwt_sh.at[kidx_v.at[j]], s_sck)
            return carry
        lax.fori_loop(0, k_kw, kw_fire, 0)

        for c_ in ld_av:
            c_.wait()
        k_av = kept_rows(aidx_v, AV_ROWS, AV_V)

        def av_fire(j, carry):
            cp(aval_v.at[j], avt_sh.at[aidx_v.at[j]], s_sca)
            return carry
        lax.fori_loop(0, k_av, av_fire, 0)

        def kw_drain(j, carry):
            pltpu.make_async_copy(kval_v.at[j], kwt_sh.at[kidx_v.at[j]],
                                  s_sck).wait()
            return carry
        lax.fori_loop(0, k_kw, kw_drain, 0)
        plsc.subcore_barrier()

        for c_ in ld_q:
            c_.wait()
        gats = [cp(kwt_sh.at[tok_v.at[j]], toko_v.at[j], s_g)
                for j in range(TOK_ROWS)]

        def av_drain(j, carry):
            pltpu.make_async_copy(aval_v.at[j], avt_sh.at[aidx_v.at[j]],
                                  s_sca).wait()
            return carry
        lax.fori_loop(0, k_av, av_drain, 0)
        plsc.subcore_barrier()
        gats.append(cp(avt_sh.at[art_v.at[0]], arto_v.at[0], s_g))
        for c_ in gats:
            c_.wait()
        pltpu.sync_copy(toko_v, otok_hbm.at[wid])
        pltpu.sync_copy(arto_v, oart_hbm.at[wid])

    return k


_sc_lookup = _build_sc_kernel()


@jax.jit
def kernel(token_ids, article_ids, keyword_vocab, article_vocab):
    tok3d = token_ids.reshape(32, TOK_ROWS, 128)
    art3d = article_ids.reshape(32, 1, 128)
    kwv = jnp.pad(keyword_vocab, (0, KW_PAD - KW_V),
                  constant_values=KW_SENT).reshape(NS, KW_ROWS, 128)
    avv = jnp.pad(article_vocab, (0, AV_PAD - AV_V),
                  constant_values=AV_SENT).reshape(NS, AV_ROWS, 128)
    kw_ramp = (jnp.arange(KW_PAD, dtype=jnp.int32) + 2).reshape(NS, KW_ROWS, 128)
    av_ramp = (jnp.arange(AV_PAD, dtype=jnp.int32) + 1).reshape(NS, AV_ROWS, 128)
    ones = jnp.ones((KW_INIT // NS,), jnp.int32)
    zeros = jnp.zeros((AV_INIT // NS,), jnp.int32)
    otok, oart = _sc_lookup(tok3d, art3d, kwv, avv, kw_ramp, av_ramp,
                            ones, zeros)
    return otok.reshape(4096, 50), oart.reshape(4096)


# speedup vs baseline: 641.5917x; 1.0404x over previous
import functools

import jax
import jax.numpy as jnp
from jax import lax
from jax.experimental import pallas as pl
from jax.experimental.pallas import tpu as pltpu
from jax.experimental.pallas import tpu_sc as plsc

L = 16
NC = 2
NS = 16

KW_V = 50000
AV_V = 100000
KW_PAD = 51200
AV_PAD = 100352
KW_ROWS = 25
AV_ROWS = 49
KW_TAB = 100352
AV_TAB = 200704
KW_INIT = 50176
AV_INIT = 100352
TOK_ROWS = 50

KW_SENT = KW_TAB - 1
AV_SENT = AV_TAB - 1


def _build_sc_kernel():
    mesh = plsc.VectorSubcoreMesh(core_axis_name="c", subcore_axis_name="s")

    @functools.partial(
        pl.kernel,
        out_type=[
            jax.ShapeDtypeStruct((32, TOK_ROWS, 128), jnp.int32),
            jax.ShapeDtypeStruct((32, 1, 128), jnp.int32),
        ],
        mesh=mesh,
        scratch_types=[
            pltpu.VMEM_SHARED((KW_TAB,), jnp.int32),
            pltpu.VMEM_SHARED((AV_TAB,), jnp.int32),
            pltpu.VMEM((KW_ROWS, 128), jnp.int32),
            pltpu.VMEM((KW_ROWS, 128), jnp.int32),
            pltpu.VMEM((AV_ROWS, 128), jnp.int32),
            pltpu.VMEM((AV_ROWS, 128), jnp.int32),
            pltpu.VMEM((TOK_ROWS, 128), jnp.int32),
            pltpu.VMEM((TOK_ROWS, 128), jnp.int32),
            pltpu.VMEM((1, 128), jnp.int32),
            pltpu.VMEM((1, 128), jnp.int32),
            pltpu.VMEM((KW_INIT // NS,), jnp.int32),
            pltpu.VMEM((AV_INIT // NS,), jnp.int32),
            pltpu.SemaphoreType.DMA,
            pltpu.SemaphoreType.DMA,
            pltpu.SemaphoreType.DMA,
            pltpu.SemaphoreType.DMA,
            pltpu.SemaphoreType.DMA,
            pltpu.SemaphoreType.DMA,
            pltpu.SemaphoreType.DMA,
            pltpu.SemaphoreType.DMA,
        ],
    )
    def k(tok_hbm, art_hbm, kw_hbm, av_hbm, kwv_hbm, avv_hbm, ones_hbm,
          zeros_hbm, otok_hbm, oart_hbm,
          kwt_sh, avt_sh, kidx_v, kval_v, aidx_v, aval_v,
          tok_v, toko_v, art_v, arto_v, ones_v, zeros_v,
          s_kw, s_av, s_q, s_stage, s_init, s_sck, s_sca, s_g):
        s = lax.axis_index("s")
        wid = lax.axis_index("c") * NS + s

        cp = pltpu.async_copy
        ld_kw = [cp(kw_hbm.at[s], kidx_v, s_kw),
                 cp(kwv_hbm.at[s], kval_v, s_kw)]
        ld_av = [cp(av_hbm.at[s], aidx_v, s_av),
                 cp(avv_hbm.at[s], aval_v, s_av)]
        ld_q = [cp(tok_hbm.at[wid], tok_v, s_q),
                cp(art_hbm.at[wid], art_v, s_q)]
        st1 = cp(ones_hbm, ones_v, s_stage)
        st2 = cp(zeros_hbm, zeros_v, s_stage)
        st1.wait()
        st2.wait()
        inits = [
            cp(ones_v,
               kwt_sh.at[pl.ds(s * (KW_INIT // NS), KW_INIT // NS)], s_init),
            cp(zeros_v,
               avt_sh.at[pl.ds(s * (AV_INIT // NS), AV_INIT // NS)], s_init),
        ]
        for c_ in inits:
            c_.wait()
        plsc.subcore_barrier()

        def kept_rows(idx_ref, n_rows, bound):
            def cnt(j, acc):
                head = idx_ref[j, pl.ds(0, L)][0]
                return acc + (head < bound).astype(jnp.int32)
            return lax.fori_loop(0, n_rows, cnt, jnp.int32(0))

        for c_ in ld_kw:
            c_.wait()
        k_kw = kept_rows(kidx_v, KW_ROWS, KW_V)

        def kw_fire(j, carry):
            cp(kval_v.at[j], kwt_sh.at[kidx_v.at[j]], s_sck)
            return carry
        lax.fori_loop(0, k_kw, kw_fire, 0)

        for c_ in ld_av:
            c_.wait()
        k_av = kept_rows(aidx_v, AV_ROWS, AV_V)

        def av_fire(j, carry):
            cp(aval_v.at[j], avt_sh.at[aidx_v.at[j]], s_sca)
            return carry
        lax.fori_loop(0, k_av, av_fire, 0)

        def kw_drain(j, carry):
            pltpu.make_async_copy(kval_v.at[j], kwt_sh.at[kidx_v.at[j]],
                                  s_sck).wait()
            return carry
        lax.fori_loop(0, k_kw, kw_drain, 0)
        plsc.subcore_barrier()

        for c_ in ld_q:
            c_.wait()
        gats = [cp(kwt_sh.at[tok_v.at[j]], toko_v.at[j], s_g)
                for j in range(TOK_ROWS)]

        def av_drain(j, carry):
            pltpu.make_async_copy(aval_v.at[j], avt_sh.at[aidx_v.at[j]],
                                  s_sca).wait()
            return carry
        lax.fori_loop(0, k_av, av_drain, 0)
        plsc.subcore_barrier()
        gats.append(cp(avt_sh.at[art_v.at[0]], arto_v.at[0], s_g))
        for c_ in gats:
            c_.wait()
        pltpu.sync_copy(toko_v, otok_hbm.at[wid])
        pltpu.sync_copy(arto_v, oart_hbm.at[wid])

    return k


_sc_lookup = _build_sc_kernel()


@jax.jit
def kernel(token_ids, article_ids, keyword_vocab, article_vocab):
    tok3d = token_ids.reshape(32, TOK_ROWS, 128)
    art3d = article_ids.reshape(32, 1, 128)
    def spread(x):
        return x.reshape(-1, NS, 128).transpose(1, 0, 2)

    kwv = spread(jnp.pad(keyword_vocab, (0, KW_PAD - KW_V),
                         constant_values=KW_SENT))
    avv = spread(jnp.pad(article_vocab, (0, AV_PAD - AV_V),
                         constant_values=AV_SENT))
    kw_ramp = spread(jnp.arange(KW_PAD, dtype=jnp.int32) + 2)
    av_ramp = spread(jnp.arange(AV_PAD, dtype=jnp.int32) + 1)
    ones = jnp.ones((KW_INIT // NS,), jnp.int32)
    zeros = jnp.zeros((AV_INIT // NS,), jnp.int32)
    otok, oart = _sc_lookup(tok3d, art3d, kwv, avv, kw_ramp, av_ramp,
                            ones, zeros)
    return otok.reshape(4096, 50), oart.reshape(4096)


# speedup vs baseline: 730.0423x; 1.1379x over previous
import functools

import jax
import jax.numpy as jnp
import numpy as np
from jax import lax
from jax.experimental import pallas as pl
from jax.experimental.pallas import tpu as pltpu
from jax.experimental.pallas import tpu_sc as plsc

L = 16
NC = 2
NS = 16
NW = NC * NS

KW_V = 50000
AV_V = 100000
KW_ROWS = 25
AV_ROWS = 49
KW_TAB = 100352
AV_TAB = 200704
KW_INIT = 50176
AV_INIT = 100352
KW_LAST = (KW_V + 127) // 128 - 1
AV_LAST = (AV_V + 127) // 128 - 1
TOK_BLK = 128


def _ramp(v, rows_per_tile, last_row, off):
    out = np.zeros((NS, rows_per_tile, 128), np.int32)
    for s_ in range(NS):
        for j in range(rows_per_tile):
            start = min((16 * j + s_) * 128, v - 128)
            out[s_, j, :] = np.arange(start, start + 128) + off
    return out


_KW_RAMP = _ramp(KW_V, KW_ROWS, KW_LAST, 2)
_AV_RAMP = _ramp(AV_V, AV_ROWS, AV_LAST, 1)
_ONES = np.ones((KW_INIT // NS,), np.int32)
_ZEROS = np.zeros((AV_INIT // NS,), np.int32)


def _build_sc_kernel():
    mesh = plsc.VectorSubcoreMesh(core_axis_name="c", subcore_axis_name="s")

    @functools.partial(
        pl.kernel,
        out_type=[
            jax.ShapeDtypeStruct((4096, 50), jnp.int32),
            jax.ShapeDtypeStruct((4096,), jnp.int32),
        ],
        mesh=mesh,
        scratch_types=[
            pltpu.VMEM_SHARED((KW_TAB,), jnp.int32),
            pltpu.VMEM_SHARED((AV_TAB,), jnp.int32),
            pltpu.VMEM((KW_ROWS, 128), jnp.int32),
            pltpu.VMEM((KW_ROWS, 128), jnp.int32),
            pltpu.VMEM((AV_ROWS, 128), jnp.int32),
            pltpu.VMEM((AV_ROWS, 128), jnp.int32),
            pltpu.VMEM((TOK_BLK, 50), jnp.int32),
            pltpu.VMEM((TOK_BLK, 50), jnp.int32),
            pltpu.VMEM((1, 128), jnp.int32),
            pltpu.VMEM((1, 128), jnp.int32),
            pltpu.VMEM((KW_INIT // NS,), jnp.int32),
            pltpu.VMEM((AV_INIT // NS,), jnp.int32),
            pltpu.SemaphoreType.DMA,
            pltpu.SemaphoreType.DMA,
            pltpu.SemaphoreType.DMA,
            pltpu.SemaphoreType.DMA,
            pltpu.SemaphoreType.DMA,
            pltpu.SemaphoreType.DMA,
            pltpu.SemaphoreType.DMA,
            pltpu.SemaphoreType.DMA,
        ],
    )
    def k(tok_hbm, art_hbm, kw_hbm, av_hbm, kwr_hbm, avr_hbm, ones_hbm,
          zeros_hbm, otok_hbm, oart_hbm,
          kwt_sh, avt_sh, kidx_v, kval_v, aidx_v, aval_v,
          tok_v, toko_v, art_v, arto_v, ones_v, zeros_v,
          s_kw, s_av, s_q, s_stage, s_init, s_sck, s_sca, s_g):
        s = lax.axis_index("s")
        wid = lax.axis_index("c") * NS + s
        n_kw = (KW_LAST - s) // NS + 1
        n_av = (AV_LAST - s) // NS + 1

        cp = pltpu.async_copy

        def row_load(vocab_hbm, dst_ref, v, sem):
            def fire(j, carry):
                start = jnp.minimum((NS * j + s) * 128, v - 128)
                cp(vocab_hbm.at[pl.ds(start, 128)], dst_ref.at[j], sem)
                return carry
            return fire

        lax.fori_loop(0, n_kw, row_load(kw_hbm, kidx_v, KW_V, s_kw), 0)
        lax.fori_loop(0, n_av, row_load(av_hbm, aidx_v, AV_V, s_av), 0)
        ld_q = [cp(kwr_hbm.at[s], kval_v, s_q),
                cp(avr_hbm.at[s], aval_v, s_q),
                cp(tok_hbm.at[pl.ds(wid * TOK_BLK, TOK_BLK)], tok_v, s_q),
                cp(art_hbm.at[pl.ds(wid * 128, 128)], art_v.at[0], s_q)]
        st1 = cp(ones_hbm, ones_v, s_stage)
        st2 = cp(zeros_hbm, zeros_v, s_stage)
        st1.wait()
        st2.wait()
        inits = [
            cp(ones_v,
               kwt_sh.at[pl.ds(s * (KW_INIT // NS), KW_INIT // NS)], s_init),
            cp(zeros_v,
               avt_sh.at[pl.ds(s * (AV_INIT // NS), AV_INIT // NS)], s_init),
        ]
        for c_ in inits:
            c_.wait()
        plsc.subcore_barrier()

        def row_drain(vocab_hbm, dst_ref, v, sem):
            def drain(j, carry):
                start = jnp.minimum((NS * j + s) * 128, v - 128)
                pltpu.make_async_copy(vocab_hbm.at[pl.ds(start, 128)],
                                      dst_ref.at[j], sem).wait()
                return carry
            return drain

        def kept_rows(idx_ref, n_rows, bound):
            def cnt(j, acc):
                head = idx_ref[j, pl.ds(0, L)][0]
                return acc + (head < bound).astype(jnp.int32)
            return lax.fori_loop(0, n_rows, cnt, jnp.int32(0))

        lax.fori_loop(0, n_kw, row_drain(kw_hbm, kidx_v, KW_V, s_kw), 0)
        k_kw = kept_rows(kidx_v, n_kw, KW_V)

        def kw_fire(j, carry):
            cp(kval_v.at[j], kwt_sh.at[kidx_v.at[j]], s_sck)
            return carry
        lax.fori_loop(0, k_kw, kw_fire, 0)

        lax.fori_loop(0, n_av, row_drain(av_hbm, aidx_v, AV_V, s_av), 0)
        k_av = kept_rows(aidx_v, n_av, AV_V)

        def av_fire(j, carry):
            cp(aval_v.at[j], avt_sh.at[aidx_v.at[j]], s_sca)
            return carry
        lax.fori_loop(0, k_av, av_fire, 0)

        def kw_drain(j, carry):
            pltpu.make_async_copy(kval_v.at[j], kwt_sh.at[kidx_v.at[j]],
                                  s_sck).wait()
            return carry
        lax.fori_loop(0, k_kw, kw_drain, 0)
        plsc.subcore_barrier()

        for c_ in ld_q:
            c_.wait()
        gats = [cp(kwt_sh.at[tok_v.at[r]], toko_v.at[r], s_g)
                for r in range(TOK_BLK)]

        def av_drain(j, carry):
            pltpu.make_async_copy(aval_v.at[j], avt_sh.at[aidx_v.at[j]],
                                  s_sca).wait()
            return carry
        lax.fori_loop(0, k_av, av_drain, 0)
        plsc.subcore_barrier()
        gats.append(cp(avt_sh.at[art_v.at[0]], arto_v.at[0], s_g))
        for c_ in gats:
            c_.wait()
        pltpu.sync_copy(toko_v, otok_hbm.at[pl.ds(wid * TOK_BLK, TOK_BLK)])
        pltpu.sync_copy(arto_v.at[0], oart_hbm.at[pl.ds(wid * 128, 128)])

    return k


_sc_lookup = _build_sc_kernel()


@jax.jit
def kernel(token_ids, article_ids, keyword_vocab, article_vocab):
    otok, oart = _sc_lookup(token_ids, article_ids, keyword_vocab,
                            article_vocab, jnp.asarray(_KW_RAMP),
                            jnp.asarray(_AV_RAMP), jnp.asarray(_ONES),
                            jnp.asarray(_ZEROS))
    return otok, oart


# speedup vs baseline: 742.2420x; 1.0167x over previous
import functools

import jax
import jax.numpy as jnp
import numpy as np
from jax import lax
from jax.experimental import pallas as pl
from jax.experimental.pallas import tpu as pltpu
from jax.experimental.pallas import tpu_sc as plsc

L = 16
NC = 2
NS = 16
NW = NC * NS

KW_V = 50000
AV_V = 100000
KW_ROWS = 25
AV_ROWS = 49
KW_TAB = 100352
AV_TAB = 200704
KW_INIT = 50176
AV_INIT = 100352
KW_LAST = (KW_V + 127) // 128 - 1
AV_LAST = (AV_V + 127) // 128 - 1
TOK_BLK = 128


def _ramp(v, rows_per_tile, last_row, off):
    out = np.zeros((NS, rows_per_tile, 128), np.int32)
    for s_ in range(NS):
        for j in range(rows_per_tile):
            start = min((16 * j + s_) * 128, v - 128)
            out[s_, j, :] = np.arange(start, start + 128) + off
    return out


_KW_RAMP = _ramp(KW_V, KW_ROWS, KW_LAST, 2)
_AV_RAMP = _ramp(AV_V, AV_ROWS, AV_LAST, 1)
_ONES = np.ones((KW_INIT // NS,), np.int32)
_ZEROS = np.zeros((AV_INIT // NS,), np.int32)


def _build_sc_kernel():
    mesh = plsc.VectorSubcoreMesh(core_axis_name="c", subcore_axis_name="s")

    @functools.partial(
        pl.kernel,
        out_type=[
            jax.ShapeDtypeStruct((4096, 50), jnp.int32),
            jax.ShapeDtypeStruct((4096,), jnp.int32),
        ],
        mesh=mesh,
        scratch_types=[
            pltpu.VMEM_SHARED((KW_TAB,), jnp.int32),
            pltpu.VMEM_SHARED((AV_TAB,), jnp.int32),
            pltpu.VMEM((KW_ROWS, 128), jnp.int32),
            pltpu.VMEM((KW_ROWS, 128), jnp.int32),
            pltpu.VMEM((AV_ROWS, 128), jnp.int32),
            pltpu.VMEM((AV_ROWS, 128), jnp.int32),
            pltpu.VMEM((TOK_BLK, 50), jnp.int32),
            pltpu.VMEM((TOK_BLK, 50), jnp.int32),
            pltpu.VMEM((1, 128), jnp.int32),
            pltpu.VMEM((1, 128), jnp.int32),
            pltpu.VMEM((KW_INIT // NS,), jnp.int32),
            pltpu.VMEM((AV_INIT // NS,), jnp.int32),
            pltpu.SemaphoreType.DMA,
            pltpu.SemaphoreType.DMA,
            pltpu.SemaphoreType.DMA,
            pltpu.SemaphoreType.DMA,
            pltpu.SemaphoreType.DMA,
            pltpu.SemaphoreType.DMA,
            pltpu.SemaphoreType.DMA,
            pltpu.SemaphoreType.DMA,
        ],
    )
    def k(tok_hbm, art_hbm, kw_hbm, av_hbm, kwr_hbm, avr_hbm, ones_hbm,
          zeros_hbm, otok_hbm, oart_hbm,
          kwt_sh, avt_sh, kidx_v, kval_v, aidx_v, aval_v,
          tok_v, toko_v, art_v, arto_v, ones_v, zeros_v,
          s_kw, s_av, s_q, s_stage, s_init, s_sck, s_sca, s_g):
        s = lax.axis_index("s")
        wid = lax.axis_index("c") * NS + s
        n_kw = (KW_LAST - s) // NS + 1
        n_av = (AV_LAST - s) // NS + 1

        cp = pltpu.async_copy

        def row_load(vocab_hbm, dst_ref, v, sem):
            def fire(j, carry):
                start = jnp.minimum((NS * j + s) * 128, v - 128)
                cp(vocab_hbm.at[pl.ds(start, 128)], dst_ref.at[j], sem)
                return carry
            return fire

        lax.fori_loop(0, n_kw, row_load(kw_hbm, kidx_v, KW_V, s_kw), 0)
        lax.fori_loop(0, n_av, row_load(av_hbm, aidx_v, AV_V, s_av), 0)
        ld_q = [cp(kwr_hbm.at[s], kval_v, s_q),
                cp(avr_hbm.at[s], aval_v, s_q),
                cp(tok_hbm.at[pl.ds(wid * TOK_BLK, TOK_BLK)], tok_v, s_q),
                cp(art_hbm.at[pl.ds(wid * 128, 128)], art_v.at[0], s_q)]
        st1 = cp(ones_hbm, ones_v, s_stage)
        st2 = cp(zeros_hbm, zeros_v, s_stage)
        st1.wait()
        st2.wait()
        inits = [
            cp(ones_v,
               kwt_sh.at[pl.ds(s * (KW_INIT // NS), KW_INIT // NS)], s_init),
            cp(zeros_v,
               avt_sh.at[pl.ds(s * (AV_INIT // NS), AV_INIT // NS)], s_init),
        ]
        for c_ in inits:
            c_.wait()
        plsc.subcore_barrier()

        def row_drain(vocab_hbm, dst_ref, v, sem):
            def drain(j, carry):
                start = jnp.minimum((NS * j + s) * 128, v - 128)
                pltpu.make_async_copy(vocab_hbm.at[pl.ds(start, 128)],
                                      dst_ref.at[j], sem).wait()
                return carry
            return drain

        def kept_rows(idx_ref, n_rows, bound):
            def cnt(j, acc):
                head = idx_ref[j, pl.ds(0, L)][0]
                return acc + (head < bound).astype(jnp.int32)
            return lax.fori_loop(0, n_rows, cnt, jnp.int32(0))

        lax.fori_loop(0, n_kw, row_drain(kw_hbm, kidx_v, KW_V, s_kw), 0)
        k_kw = kept_rows(kidx_v, n_kw, KW_V)

        def kw_fire(j, carry):
            cp(kval_v.at[j], kwt_sh.at[kidx_v.at[j]], s_sck)
            return carry
        lax.fori_loop(0, k_kw, kw_fire, 0)

        lax.fori_loop(0, n_av, row_drain(av_hbm, aidx_v, AV_V, s_av), 0)
        k_av = kept_rows(aidx_v, n_av, AV_V)

        def av_fire(j, carry):
            cp(aval_v.at[j], avt_sh.at[aidx_v.at[j]], s_sca)
            return carry
        lax.fori_loop(0, k_av, av_fire, 0)

        def kw_drain(j, carry):
            pltpu.make_async_copy(kval_v.at[j], kwt_sh.at[kidx_v.at[j]],
                                  s_sck).wait()
            return carry
        lax.fori_loop(0, k_kw, kw_drain, 0)
        plsc.subcore_barrier()

        for c_ in ld_q:
            c_.wait()

        def tok_fire(r, carry):
            cp(kwt_sh.at[tok_v.at[r]], toko_v.at[r], s_g)
            return carry
        lax.fori_loop(0, TOK_BLK, tok_fire, 0)

        def av_drain(j, carry):
            pltpu.make_async_copy(aval_v.at[j], avt_sh.at[aidx_v.at[j]],
                                  s_sca).wait()
            return carry
        lax.fori_loop(0, k_av, av_drain, 0)
        plsc.subcore_barrier()
        ga = cp(avt_sh.at[art_v.at[0]], arto_v.at[0], s_stage)

        def tok_drain(r, carry):
            pltpu.make_async_copy(kwt_sh.at[tok_v.at[r]], toko_v.at[r],
                                  s_g).wait()
            return carry
        lax.fori_loop(0, TOK_BLK, tok_drain, 0)
        ga.wait()
        pltpu.sync_copy(toko_v, otok_hbm.at[pl.ds(wid * TOK_BLK, TOK_BLK)])
        pltpu.sync_copy(arto_v.at[0], oart_hbm.at[pl.ds(wid * 128, 128)])

    return k


_sc_lookup = _build_sc_kernel()


@jax.jit
def kernel(token_ids, article_ids, keyword_vocab, article_vocab):
    otok, oart = _sc_lookup(token_ids, article_ids, keyword_vocab,
                            article_vocab, jnp.asarray(_KW_RAMP),
                            jnp.asarray(_AV_RAMP), jnp.asarray(_ONES),
                            jnp.asarray(_ZEROS))
    return otok, oart


# speedup vs baseline: 813.9231x; 1.0966x over previous
import functools

import jax
import jax.numpy as jnp
import numpy as np
from jax import lax
from jax.experimental import pallas as pl
from jax.experimental.pallas import tpu as pltpu
from jax.experimental.pallas import tpu_sc as plsc

L = 16
NC = 2
NS = 16
NW = NC * NS

KW_V = 50000
AV_V = 100000
KW_ROWS = 25
AV_ROWS = 49
KW_TAB = 100352
AV_TAB = 200704
KW_INIT = 50176
AV_INIT = 100352
KW_LAST = (KW_V + 127) // 128 - 1
AV_LAST = (AV_V + 127) // 128 - 1
TOK_BLK = 128


def _ramp(v, rows_per_tile, last_row, off):
    out = np.zeros((NS, rows_per_tile, 128), np.int32)
    for s_ in range(NS):
        for j in range(rows_per_tile):
            start = min((16 * j + s_) * 128, v - 128)
            out[s_, j, :] = np.arange(start, start + 128) + off
    return out


_KW_RAMP = _ramp(KW_V, KW_ROWS, KW_LAST, 2)
_AV_RAMP = _ramp(AV_V, AV_ROWS, AV_LAST, 1)


def _build_sc_kernel():
    mesh = plsc.VectorSubcoreMesh(core_axis_name="c", subcore_axis_name="s")

    @functools.partial(
        pl.kernel,
        out_type=[
            jax.ShapeDtypeStruct((4096, 50), jnp.int32),
            jax.ShapeDtypeStruct((4096,), jnp.int32),
        ],
        mesh=mesh,
        scratch_types=[
            pltpu.VMEM_SHARED((KW_TAB,), jnp.int32),
            pltpu.VMEM_SHARED((AV_TAB,), jnp.int32),
            pltpu.VMEM((KW_ROWS, 128), jnp.int32),
            pltpu.VMEM((KW_ROWS, 128), jnp.int32),
            pltpu.VMEM((AV_ROWS, 128), jnp.int32),
            pltpu.VMEM((AV_ROWS, 128), jnp.int32),
            pltpu.VMEM((TOK_BLK, 50), jnp.int32),
            pltpu.VMEM((TOK_BLK, 50), jnp.int32),
            pltpu.VMEM((1, 128), jnp.int32),
            pltpu.VMEM((1, 128), jnp.int32),
            pltpu.VMEM((KW_INIT // NS,), jnp.int32),
            pltpu.VMEM((AV_INIT // NS,), jnp.int32),
            pltpu.SemaphoreType.DMA,
            pltpu.SemaphoreType.DMA,
            pltpu.SemaphoreType.DMA,
            pltpu.SemaphoreType.DMA,
            pltpu.SemaphoreType.DMA,
            pltpu.SemaphoreType.DMA,
            pltpu.SemaphoreType.DMA,
            pltpu.SemaphoreType.DMA,
        ],
    )
    def k(tok_hbm, art_hbm, kw_hbm, av_hbm, kwr_hbm, avr_hbm,
          otok_hbm, oart_hbm,
          kwt_sh, avt_sh, kidx_v, kval_v, aidx_v, aval_v,
          tok_v, toko_v, art_v, arto_v, ones_v, zeros_v,
          s_kw, s_av, s_q, s_stage, s_init, s_sck, s_sca, s_g):
        s = lax.axis_index("s")
        wid = lax.axis_index("c") * NS + s
        n_kw = (KW_LAST - s) // NS + 1
        n_av = (AV_LAST - s) // NS + 1

        cp = pltpu.async_copy

        def row_load(vocab_hbm, dst_ref, v, sem):
            def fire(j, carry):
                start = jnp.minimum((NS * j + s) * 128, v - 128)
                cp(vocab_hbm.at[pl.ds(start, 128)], dst_ref.at[j], sem)
                return carry
            return fire

        lax.fori_loop(0, n_kw, row_load(kw_hbm, kidx_v, KW_V, s_kw), 0)
        lax.fori_loop(0, n_av, row_load(av_hbm, aidx_v, AV_V, s_av), 0)
        ld_q = [cp(kwr_hbm.at[s], kval_v, s_q),
                cp(avr_hbm.at[s], aval_v, s_q),
                cp(tok_hbm.at[pl.ds(wid * TOK_BLK, TOK_BLK)], tok_v, s_q),
                cp(art_hbm.at[pl.ds(wid * 128, 128)], art_v.at[0], s_q)]

        def fill(ref, n, val):
            def st(i, carry):
                ref[pl.ds(i * L, L)] = jnp.full((L,), val, jnp.int32)
                return carry
            lax.fori_loop(0, n // L, st, 0)

        fill(ones_v, KW_INIT // NS, 1)
        fill(zeros_v, AV_INIT // NS, 0)
        inits = [
            cp(ones_v,
               kwt_sh.at[pl.ds(s * (KW_INIT // NS), KW_INIT // NS)], s_init),
            cp(zeros_v,
               avt_sh.at[pl.ds(s * (AV_INIT // NS), AV_INIT // NS)], s_init),
        ]

        def row_drain(vocab_hbm, dst_ref, v, sem):
            def drain(j, carry):
                start = jnp.minimum((NS * j + s) * 128, v - 128)
                pltpu.make_async_copy(vocab_hbm.at[pl.ds(start, 128)],
                                      dst_ref.at[j], sem).wait()
                return carry
            return drain

        def kept_rows(idx_ref, n_rows, bound):
            def cnt(j, acc):
                head = idx_ref[j, pl.ds(0, L)][0]
                return acc + (head < bound).astype(jnp.int32)
            return lax.fori_loop(0, n_rows, cnt, jnp.int32(0))

        lax.fori_loop(0, n_kw, row_drain(kw_hbm, kidx_v, KW_V, s_kw), 0)
        k_kw = kept_rows(kidx_v, n_kw, KW_V)
        lax.fori_loop(0, n_av, row_drain(av_hbm, aidx_v, AV_V, s_av), 0)
        k_av = kept_rows(aidx_v, n_av, AV_V)
        for c_ in inits:
            c_.wait()
        plsc.subcore_barrier()

        def kw_fire(j, carry):
            cp(kval_v.at[j], kwt_sh.at[kidx_v.at[j]], s_sck)
            return carry
        lax.fori_loop(0, k_kw, kw_fire, 0)

        def av_fire(j, carry):
            cp(aval_v.at[j], avt_sh.at[aidx_v.at[j]], s_sca)
            return carry
        lax.fori_loop(0, k_av, av_fire, 0)

        def kw_drain(j, carry):
            pltpu.make_async_copy(kval_v.at[j], kwt_sh.at[kidx_v.at[j]],
                                  s_sck).wait()
            return carry
        lax.fori_loop(0, k_kw, kw_drain, 0)
        plsc.subcore_barrier()

        for c_ in ld_q:
            c_.wait()

        def tok_fire(r, carry):
            cp(kwt_sh.at[tok_v.at[r]], toko_v.at[r], s_g)
            return carry
        lax.fori_loop(0, TOK_BLK, tok_fire, 0)

        def av_drain(j, carry):
            pltpu.make_async_copy(aval_v.at[j], avt_sh.at[aidx_v.at[j]],
                                  s_sca).wait()
            return carry
        lax.fori_loop(0, k_av, av_drain, 0)
        plsc.subcore_barrier()
        ga = cp(avt_sh.at[art_v.at[0]], arto_v.at[0], s_stage)

        def tok_drain(r, carry):
            pltpu.make_async_copy(kwt_sh.at[tok_v.at[r]], toko_v.at[r],
                                  s_g).wait()
            return carry
        lax.fori_loop(0, TOK_BLK, tok_drain, 0)
        ga.wait()
        w1 = cp(toko_v, otok_hbm.at[pl.ds(wid * TOK_BLK, TOK_BLK)], s_kw)
        w2 = cp(arto_v.at[0], oart_hbm.at[pl.ds(wid * 128, 128)], s_av)
        w1.wait()
        w2.wait()

    return k


_sc_lookup = _build_sc_kernel()


@jax.jit
def kernel(token_ids, article_ids, keyword_vocab, article_vocab):
    otok, oart = _sc_lookup(token_ids, article_ids, keyword_vocab,
                            article_vocab, jnp.asarray(_KW_RAMP),
                            jnp.asarray(_AV_RAMP))
    return otok, oart
